# Initial kernel scaffold; baseline (speedup 1.0000x reference)
import functools

import jax
import jax.numpy as jnp
from jax import lax
from jax.experimental import pallas as pl
from jax.experimental.pallas import tpu as pltpu
from jax.experimental.pallas import tpu_sc as plsc

NC = 2
NS = 16
NW = NC * NS

CHUNK = 80


def _sc_scatter_add(n_nodes, n_edges, d):
    ept = n_edges // NW
    iters = ept // CHUNK
    rows_per_tile = n_nodes // NS

    mesh = plsc.VectorSubcoreMesh(core_axis_name="c", subcore_axis_name="s")

    @functools.partial(
        pl.kernel,
        out_type=jax.ShapeDtypeStruct((NC, n_nodes, d), jnp.float32),
        mesh=mesh,
        scratch_types=[
            pltpu.VMEM((CHUNK,), jnp.int32),
            pltpu.VMEM((CHUNK,), jnp.int32),
            pltpu.VMEM((CHUNK, d), jnp.float32),
            pltpu.VMEM((CHUNK, d), jnp.float32),
            pltpu.VMEM_SHARED((n_nodes, d), jnp.float32),
            pltpu.SemaphoreType.DMA,
            pltpu.SemaphoreType.DMA,
            pltpu.SemaphoreType.DMA,
            pltpu.SemaphoreType.DMA,
        ],
    )
    def scatter_kernel(dst_hbm, ea_hbm, zeros_hbm, out_hbm,
                       idx0, idx1, row0, row1, acc, si0, si1, sr0, sr1):
        c = lax.axis_index("c")
        s = lax.axis_index("s")
        wid = c * NS + s
        base = wid * ept

        r0 = s * rows_per_tile
        pltpu.sync_copy(zeros_hbm.at[pl.ds(r0, rows_per_tile)],
                        acc.at[pl.ds(r0, rows_per_tile)])
        plsc.subcore_barrier()

        def issue(i, idx_ref, row_ref, sem_i, sem_r):
            off = base + i * CHUNK
            pltpu.async_copy(dst_hbm.at[pl.ds(off, CHUNK)], idx_ref, sem_i)
            pltpu.async_copy(ea_hbm.at[pl.ds(off, CHUNK)], row_ref, sem_r)

        def wait(idx_ref, row_ref, sem_i, sem_r):
            pltpu.make_async_copy(dst_hbm.at[pl.ds(0, CHUNK)], idx_ref, sem_i).wait()
            pltpu.make_async_copy(ea_hbm.at[pl.ds(0, CHUNK)], row_ref, sem_r).wait()

        def scatter(idx_ref, row_ref):
            pltpu.sync_copy(row_ref, acc.at[idx_ref], add=True)

        issue(0, idx0, row0, si0, sr0)

        def pair(g, carry):
            i0 = 2 * g
            issue(i0 + 1, idx1, row1, si1, sr1)
            wait(idx0, row0, si0, sr0)
            scatter(idx0, row0)

            @pl.when(i0 + 2 < iters)
            def _():
                issue(i0 + 2, idx0, row0, si0, sr0)

            wait(idx1, row1, si1, sr1)
            scatter(idx1, row1)
            return carry

        lax.fori_loop(0, iters // 2, pair, 0)
        if iters % 2:
            wait(idx0, row0, si0, sr0)
            scatter(idx0, row0)

        plsc.subcore_barrier()
        pltpu.sync_copy(acc.at[pl.ds(r0, rows_per_tile)],
                        out_hbm.at[c, pl.ds(r0, rows_per_tile)])

    return scatter_kernel


def _mlp_body(x_ref, a0_ref, a1_ref, w1x_ref, w1a_ref, b1_ref,
              w2_ref, b2_ref, g_ref, bt_ref, o_ref):
    x = x_ref[...]
    a = a0_ref[...] + a1_ref[...]
    h = (jnp.dot(x, w1x_ref[...], preferred_element_type=jnp.float32,
                 precision=lax.Precision.HIGHEST)
         + jnp.dot(a, w1a_ref[...], preferred_element_type=jnp.float32,
                   precision=lax.Precision.HIGHEST)
         + b1_ref[...])
    h = h * jax.nn.sigmoid(h)
    y = (jnp.dot(h, w2_ref[...], preferred_element_type=jnp.float32,
                 precision=lax.Precision.HIGHEST)
         + b2_ref[...])
    mean = jnp.mean(y, axis=-1, keepdims=True)
    cen = y - mean
    var = jnp.mean(cen * cen, axis=-1, keepdims=True)
    o_ref[...] = cen * lax.rsqrt(var + 1e-5) * g_ref[...] + bt_ref[...] + x


def kernel(x, edge_index, edge_attr, W1, b1, W2, b2, ln_gamma, ln_beta):
    n_nodes, d = x.shape
    n_edges = edge_attr.shape[0]

    dst = jnp.asarray(edge_index[0], jnp.int32)
    zeros = jnp.zeros((n_nodes, d), jnp.float32)

    agg2 = _sc_scatter_add(n_nodes, n_edges, d)(dst, edge_attr, zeros)
    a0, a1 = agg2[0], agg2[1]

    w1x = W1[:d]
    w1a = W1[d:]

    blk = 1000
    grid = (n_nodes // blk,)
    row_spec = pl.BlockSpec((blk, d), lambda i: (i, 0))
    full = lambda shape: pl.BlockSpec(shape, lambda i: (0,) * len(shape))

    out = pl.pallas_call(
        _mlp_body,
        grid=grid,
        in_specs=[
            row_spec, row_spec, row_spec,
            full((d, d)), full((d, d)), full((1, d)),
            full((d, d)), full((1, d)), full((1, d)), full((1, d)),
        ],
        out_specs=row_spec,
        out_shape=jax.ShapeDtypeStruct((n_nodes, d), jnp.float32),
    )(x, a0, a1, w1x, w1a, b1.reshape(1, -1), W2, b2.reshape(1, -1),
      ln_gamma.reshape(1, -1), ln_beta.reshape(1, -1))
    return out


# speedup vs baseline: 5.5603x; 5.5603x over previous
import functools

import jax
import jax.numpy as jnp
from jax import lax
from jax.experimental import pallas as pl
from jax.experimental.pallas import tpu as pltpu
from jax.experimental.pallas import tpu_sc as plsc

NC = 2
NS = 16
NW = NC * NS

CHUNK = 80


def _sc_scatter_add(n_nodes, n_edges, d):
    ept = n_edges // NW
    iters = ept // CHUNK
    rows_per_tile = -(-n_nodes // (8 * NS)) * 8
    n_pad = rows_per_tile * NS

    mesh = plsc.VectorSubcoreMesh(core_axis_name="c", subcore_axis_name="s")

    @functools.partial(
        pl.kernel,
        out_type=jax.ShapeDtypeStruct((NC, n_pad, d), jnp.float32),
        mesh=mesh,
        scratch_types=[
            pltpu.VMEM((CHUNK,), jnp.int32),
            pltpu.VMEM((CHUNK,), jnp.int32),
            pltpu.VMEM((CHUNK, d), jnp.float32),
            pltpu.VMEM((CHUNK, d), jnp.float32),
            pltpu.VMEM_SHARED((n_pad, d), jnp.float32),
            pltpu.SemaphoreType.DMA,
            pltpu.SemaphoreType.DMA,
            pltpu.SemaphoreType.DMA,
            pltpu.SemaphoreType.DMA,
        ],
    )
    def scatter_kernel(dst_hbm, ea_hbm, zeros_hbm, out_hbm,
                       idx0, idx1, row0, row1, acc, si0, si1, sr0, sr1):
        c = lax.axis_index("c")
        s = lax.axis_index("s")
        wid = c * NS + s
        base = wid * ept

        r0 = s * rows_per_tile
        pltpu.sync_copy(zeros_hbm.at[pl.ds(r0, rows_per_tile)],
                        acc.at[pl.ds(r0, rows_per_tile)])
        plsc.subcore_barrier()

        def issue(i, idx_ref, row_ref, sem_i, sem_r):
            off = base + i * CHUNK
            pltpu.async_copy(dst_hbm.at[pl.ds(off, CHUNK)], idx_ref, sem_i)
            pltpu.async_copy(ea_hbm.at[pl.ds(off, CHUNK)], row_ref, sem_r)

        def wait(idx_ref, row_ref, sem_i, sem_r):
            pltpu.make_async_copy(dst_hbm.at[pl.ds(0, CHUNK)], idx_ref, sem_i).wait()
            pltpu.make_async_copy(ea_hbm.at[pl.ds(0, CHUNK)], row_ref, sem_r).wait()

        def scatter(idx_ref, row_ref):
            pltpu.sync_copy(row_ref, acc.at[idx_ref], add=True)

        issue(0, idx0, row0, si0, sr0)

        def pair(g, carry):
            i0 = 2 * g
            issue(i0 + 1, idx1, row1, si1, sr1)
            wait(idx0, row0, si0, sr0)
            scatter(idx0, row0)

            @pl.when(i0 + 2 < iters)
            def _():
                issue(i0 + 2, idx0, row0, si0, sr0)

            wait(idx1, row1, si1, sr1)
            scatter(idx1, row1)
            return carry

        lax.fori_loop(0, iters // 2, pair, 0)
        if iters % 2:
            wait(idx0, row0, si0, sr0)
            scatter(idx0, row0)

        plsc.subcore_barrier()
        pltpu.sync_copy(acc.at[pl.ds(r0, rows_per_tile)],
                        out_hbm.at[c, pl.ds(r0, rows_per_tile)])

    return scatter_kernel


def _mlp_body(x_ref, a0_ref, a1_ref, w1x_ref, w1a_ref, b1_ref,
              w2_ref, b2_ref, g_ref, bt_ref, o_ref):
    x = x_ref[...]
    a = a0_ref[...] + a1_ref[...]
    h = (jnp.dot(x, w1x_ref[...], preferred_element_type=jnp.float32,
                 precision=lax.Precision.HIGHEST)
         + jnp.dot(a, w1a_ref[...], preferred_element_type=jnp.float32,
                   precision=lax.Precision.HIGHEST)
         + b1_ref[...])
    h = h * jax.nn.sigmoid(h)
    y = (jnp.dot(h, w2_ref[...], preferred_element_type=jnp.float32,
                 precision=lax.Precision.HIGHEST)
         + b2_ref[...])
    mean = jnp.mean(y, axis=-1, keepdims=True)
    cen = y - mean
    var = jnp.mean(cen * cen, axis=-1, keepdims=True)
    o_ref[...] = cen * lax.rsqrt(var + 1e-5) * g_ref[...] + bt_ref[...] + x


def kernel(x, edge_index, edge_attr, W1, b1, W2, b2, ln_gamma, ln_beta):
    n_nodes, d = x.shape
    n_edges = edge_attr.shape[0]

    dst = jnp.asarray(edge_index[0], jnp.int32)
    n_pad = -(-n_nodes // (8 * NS)) * 8 * NS
    zeros = jnp.zeros((n_pad, d), jnp.float32)

    agg2 = _sc_scatter_add(n_nodes, n_edges, d)(dst, edge_attr, zeros)
    a0, a1 = agg2[0, :n_nodes], agg2[1, :n_nodes]

    w1x = W1[:d]
    w1a = W1[d:]

    blk = 1000
    grid = (n_nodes // blk,)
    row_spec = pl.BlockSpec((blk, d), lambda i: (i, 0))
    full = lambda shape: pl.BlockSpec(shape, lambda i: (0,) * len(shape))

    out = pl.pallas_call(
        _mlp_body,
        grid=grid,
        in_specs=[
            row_spec, row_spec, row_spec,
            full((d, d)), full((d, d)), full((1, d)),
            full((d, d)), full((1, d)), full((1, d)), full((1, d)),
        ],
        out_specs=row_spec,
        out_shape=jax.ShapeDtypeStruct((n_nodes, d), jnp.float32),
    )(x, a0, a1, w1x, w1a, b1.reshape(1, -1), W2, b2.reshape(1, -1),
      ln_gamma.reshape(1, -1), ln_beta.reshape(1, -1))
    return out


# speedup vs baseline: 6.5638x; 1.1805x over previous
import functools

import jax
import jax.numpy as jnp
from jax import lax
from jax.experimental import pallas as pl
from jax.experimental.pallas import tpu as pltpu
from jax.experimental.pallas import tpu_sc as plsc

NC = 2
NS = 16
NW = NC * NS

CHUNK = 80


def _sc_scatter_add(n_nodes, n_edges, d):
    ept = n_edges // NW
    iters = ept // CHUNK
    assert iters >= 6
    rows_per_tile = -(-n_nodes // (8 * NS)) * 8
    n_pad = rows_per_tile * NS

    mesh = plsc.VectorSubcoreMesh(core_axis_name="c", subcore_axis_name="s")

    def body(dst_hbm, ea_hbm, out_hbm,
             idx0, idx1, idx2, row0, row1, row2, acc,
             g0, g1, g2, s0, s1, s2):
        c = lax.axis_index("c")
        s = lax.axis_index("s")
        wid = c * NS + s
        base = wid * ept
        idxs = (idx0, idx1, idx2)
        rows = (row0, row1, row2)
        gsem = (g0, g1, g2)
        ssem = (s0, s1, s2)

        def zrow(r, carry):
            z = jnp.zeros((16,), jnp.float32)
            for cc in range(d // 16):
                row0[r, pl.ds(cc * 16, 16)] = z
            return carry
        lax.fori_loop(0, CHUNK, zrow, 0)
        r0 = s * rows_per_tile
        for j in range(rows_per_tile // CHUNK):
            pltpu.sync_copy(row0, acc.at[pl.ds(r0 + j * CHUNK, CHUNK)])
        rem_rows = rows_per_tile % CHUNK
        if rem_rows:
            pltpu.sync_copy(row0.at[pl.ds(0, rem_rows)],
                            acc.at[pl.ds(r0 + rows_per_tile - rem_rows,
                                         rem_rows)])
        plsc.subcore_barrier()

        def issue_gather(i, b):
            off = base + i * CHUNK
            pltpu.async_copy(dst_hbm.at[pl.ds(off, CHUNK)], idxs[b], gsem[b])
            pltpu.async_copy(ea_hbm.at[pl.ds(off, CHUNK)], rows[b], gsem[b])

        def wait_gather(b):
            pltpu.make_async_copy(dst_hbm.at[pl.ds(0, CHUNK)], idxs[b],
                                  gsem[b]).wait()
            pltpu.make_async_copy(ea_hbm.at[pl.ds(0, CHUNK)], rows[b],
                                  gsem[b]).wait()

        def issue_scatter(b):
            pltpu.async_copy(rows[b], acc.at[idxs[b]], ssem[b], add=True)

        def wait_scatter(b):
            pltpu.make_async_copy(rows[b], acc.at[idxs[b]], ssem[b]).wait()

        issue_gather(0, 0)
        issue_gather(1, 1)
        wait_gather(0)
        issue_scatter(0)
        issue_gather(2, 2)
        wait_gather(1)
        issue_scatter(1)
        wait_scatter(0)
        issue_gather(3, 0)
        wait_gather(2)
        issue_scatter(2)
        wait_scatter(1)
        issue_gather(4, 1)

        G = iters // 3

        def group(g, carry):
            i0 = 3 * g
            for k in range(3):
                b = k
                nb = (k + 2) % 3
                wait_gather(b)
                issue_scatter(b)
                wait_scatter(nb)
                issue_gather(i0 + k + 2, nb)
            return carry
        lax.fori_loop(1, G, group, 0)

        for i in range(3 * G, iters):
            b = i % 3
            wait_gather(b)
            issue_scatter(b)
            wait_scatter((b + 2) % 3)
        wait_scatter((iters - 1) % 3)

        plsc.subcore_barrier()
        pltpu.sync_copy(acc.at[pl.ds(r0, rows_per_tile)],
                        out_hbm.at[c, pl.ds(r0, rows_per_tile)])

    scatter_kernel = functools.partial(
        pl.kernel,
        out_type=jax.ShapeDtypeStruct((NC, n_pad, d), jnp.float32),
        mesh=mesh,
        scratch_types=[
            pltpu.VMEM((CHUNK,), jnp.int32),
            pltpu.VMEM((CHUNK,), jnp.int32),
            pltpu.VMEM((CHUNK,), jnp.int32),
            pltpu.VMEM((CHUNK, d), jnp.float32),
            pltpu.VMEM((CHUNK, d), jnp.float32),
            pltpu.VMEM((CHUNK, d), jnp.float32),
            pltpu.VMEM_SHARED((n_pad, d), jnp.float32),
            pltpu.SemaphoreType.DMA,
            pltpu.SemaphoreType.DMA,
            pltpu.SemaphoreType.DMA,
            pltpu.SemaphoreType.DMA,
            pltpu.SemaphoreType.DMA,
            pltpu.SemaphoreType.DMA,
        ],
    )(body)
    return scatter_kernel, n_pad


def _mlp_body(x_ref, a2_ref, w1x_ref, w1a_ref, b1_ref,
              w2_ref, b2_ref, g_ref, bt_ref, o_ref):
    x = x_ref[...]
    a = a2_ref[0] + a2_ref[1]
    h = (jnp.dot(x, w1x_ref[...], preferred_element_type=jnp.float32,
                 precision=lax.Precision.HIGHEST)
         + jnp.dot(a, w1a_ref[...], preferred_element_type=jnp.float32,
                   precision=lax.Precision.HIGHEST)
         + b1_ref[...])
    h = h * jax.nn.sigmoid(h)
    y = (jnp.dot(h, w2_ref[...], preferred_element_type=jnp.float32,
                 precision=lax.Precision.HIGHEST)
         + b2_ref[...])
    mean = jnp.mean(y, axis=-1, keepdims=True)
    cen = y - mean
    var = jnp.mean(cen * cen, axis=-1, keepdims=True)
    o_ref[...] = cen * lax.rsqrt(var + 1e-5) * g_ref[...] + bt_ref[...] + x


def kernel(x, edge_index, edge_attr, W1, b1, W2, b2, ln_gamma, ln_beta):
    n_nodes, d = x.shape
    n_edges = edge_attr.shape[0]

    dst = jnp.asarray(edge_index[0], jnp.int32)

    sc_fn, n_pad = _sc_scatter_add(n_nodes, n_edges, d)
    agg2 = sc_fn(dst, edge_attr)

    w1x = W1[:d]
    w1a = W1[d:]

    blk = 1000
    grid = (n_nodes // blk,)
    row_spec = pl.BlockSpec((blk, d), lambda i: (i, 0))
    full = lambda shape: pl.BlockSpec(shape, lambda i: (0,) * len(shape))

    out = pl.pallas_call(
        _mlp_body,
        grid=grid,
        in_specs=[
            row_spec,
            pl.BlockSpec((2, blk, d), lambda i: (0, i, 0)),
            full((d, d)), full((d, d)), full((1, d)),
            full((d, d)), full((1, d)), full((1, d)), full((1, d)),
        ],
        out_specs=row_spec,
        out_shape=jax.ShapeDtypeStruct((n_nodes, d), jnp.float32),
    )(x, agg2, w1x, w1a, b1.reshape(1, -1), W2, b2.reshape(1, -1),
      ln_gamma.reshape(1, -1), ln_beta.reshape(1, -1))
    return out


# speedup vs baseline: 6.8952x; 1.0505x over previous
import functools

import jax
import jax.numpy as jnp
from jax import lax
from jax.experimental import pallas as pl
from jax.experimental.pallas import tpu as pltpu
from jax.experimental.pallas import tpu_sc as plsc

NC = 2
NS = 16
NW = NC * NS

CHUNK = 80


def _sc_scatter_add(n_nodes, n_edges, d):
    ept = n_edges // NW
    iters = ept // CHUNK
    assert iters >= 6
    rows_per_tile = -(-n_nodes // (8 * NS)) * 8
    n_pad = rows_per_tile * NS

    mesh = plsc.VectorSubcoreMesh(core_axis_name="c", subcore_axis_name="s")

    def body(dst_hbm, ea_hbm, out_hbm,
             idx0, idx1, idx2, row0, row1, row2, acc,
             g0, g1, g2, s0, s1, s2):
        c = lax.axis_index("c")
        s = lax.axis_index("s")
        wid = c * NS + s
        base = wid * ept
        idxs = (idx0, idx1, idx2)
        rows = (row0, row1, row2)
        gsem = (g0, g1, g2)
        ssem = (s0, s1, s2)

        def zrow(r, carry):
            z = jnp.zeros((16,), jnp.float32)
            for cc in range(d // 16):
                row0[r, pl.ds(cc * 16, 16)] = z
            return carry
        lax.fori_loop(0, CHUNK, zrow, 0)
        r0 = s * rows_per_tile
        for j in range(rows_per_tile // CHUNK):
            pltpu.sync_copy(row0, acc.at[pl.ds(r0 + j * CHUNK, CHUNK)])
        rem_rows = rows_per_tile % CHUNK
        if rem_rows:
            pltpu.sync_copy(row0.at[pl.ds(0, rem_rows)],
                            acc.at[pl.ds(r0 + rows_per_tile - rem_rows,
                                         rem_rows)])
        plsc.subcore_barrier()

        def issue_gather(i, b):
            off = base + i * CHUNK
            pltpu.async_copy(dst_hbm.at[pl.ds(off, CHUNK)], idxs[b], gsem[b])
            pltpu.async_copy(ea_hbm.at[pl.ds(off, CHUNK)], rows[b], gsem[b])

        def wait_gather(b):
            pltpu.make_async_copy(dst_hbm.at[pl.ds(0, CHUNK)], idxs[b],
                                  gsem[b]).wait()
            pltpu.make_async_copy(ea_hbm.at[pl.ds(0, CHUNK)], rows[b],
                                  gsem[b]).wait()

        def issue_scatter(b):
            pltpu.async_copy(rows[b], acc.at[idxs[b]], ssem[b], add=True)

        def wait_scatter(b):
            pltpu.make_async_copy(rows[b], acc.at[idxs[b]], ssem[b]).wait()

        issue_gather(0, 0)
        issue_gather(1, 1)
        wait_gather(0)
        issue_scatter(0)
        issue_gather(2, 2)
        wait_gather(1)
        issue_scatter(1)
        wait_scatter(0)
        issue_gather(3, 0)
        wait_gather(2)
        issue_scatter(2)
        wait_scatter(1)
        issue_gather(4, 1)

        G = iters // 3

        def group(g, carry):
            i0 = 3 * g
            for k in range(3):
                b = k
                nb = (k + 2) % 3
                wait_gather(b)
                issue_scatter(b)
                wait_scatter(nb)
                issue_gather(i0 + k + 2, nb)
            return carry
        lax.fori_loop(1, G, group, 0)

        for i in range(3 * G, iters):
            b = i % 3
            wait_gather(b)
            issue_scatter(b)
            wait_scatter((b + 2) % 3)
        wait_scatter((iters - 1) % 3)

        plsc.subcore_barrier()
        pltpu.sync_copy(acc.at[pl.ds(r0, rows_per_tile)],
                        out_hbm.at[c, pl.ds(r0, rows_per_tile)])

    scatter_kernel = functools.partial(
        pl.kernel,
        out_type=jax.ShapeDtypeStruct((NC, n_pad, d), jnp.float32),
        mesh=mesh,
        scratch_types=[
            pltpu.VMEM((CHUNK,), jnp.int32),
            pltpu.VMEM((CHUNK,), jnp.int32),
            pltpu.VMEM((CHUNK,), jnp.int32),
            pltpu.VMEM((CHUNK, d), jnp.float32),
            pltpu.VMEM((CHUNK, d), jnp.float32),
            pltpu.VMEM((CHUNK, d), jnp.float32),
            pltpu.VMEM_SHARED((n_pad, d), jnp.float32),
            pltpu.SemaphoreType.DMA,
            pltpu.SemaphoreType.DMA,
            pltpu.SemaphoreType.DMA,
            pltpu.SemaphoreType.DMA,
            pltpu.SemaphoreType.DMA,
            pltpu.SemaphoreType.DMA,
        ],
    )(body)
    return scatter_kernel, n_pad


def _mlp_body(x_ref, a2_ref, w1_ref, b1_ref,
              w2_ref, b2_ref, g_ref, bt_ref, o_ref):
    x = x_ref[...]
    a = a2_ref[0] + a2_ref[1]
    xa = jnp.concatenate([x, a], axis=-1)
    h = (jnp.dot(xa, w1_ref[...], preferred_element_type=jnp.float32,
                 precision=lax.Precision.HIGHEST)
         + b1_ref[...])
    h = h * jax.nn.sigmoid(h)
    y = (jnp.dot(h, w2_ref[...], preferred_element_type=jnp.float32,
                 precision=lax.Precision.HIGHEST)
         + b2_ref[...])
    mean = jnp.mean(y, axis=-1, keepdims=True)
    cen = y - mean
    var = jnp.mean(cen * cen, axis=-1, keepdims=True)
    o_ref[...] = cen * lax.rsqrt(var + 1e-5) * g_ref[...] + bt_ref[...] + x


def kernel(x, edge_index, edge_attr, W1, b1, W2, b2, ln_gamma, ln_beta):
    n_nodes, d = x.shape
    n_edges = edge_attr.shape[0]

    dst = jnp.asarray(edge_index[0], jnp.int32)

    sc_fn, n_pad = _sc_scatter_add(n_nodes, n_edges, d)
    agg2 = sc_fn(dst, edge_attr)

    blk = 1000
    grid = (n_nodes // blk,)
    row_spec = pl.BlockSpec((blk, d), lambda i: (i, 0))
    full = lambda shape: pl.BlockSpec(shape, lambda i: (0,) * len(shape))

    out = pl.pallas_call(
        _mlp_body,
        grid=grid,
        in_specs=[
            row_spec,
            pl.BlockSpec((2, blk, d), lambda i: (0, i, 0)),
            full((2 * d, d)), full((1, d)),
            full((d, d)), full((1, d)), full((1, d)), full((1, d)),
        ],
        out_specs=row_spec,
        out_shape=jax.ShapeDtypeStruct((n_nodes, d), jnp.float32),
    )(x, agg2, W1, b1.reshape(1, -1), W2, b2.reshape(1, -1),
      ln_gamma.reshape(1, -1), ln_beta.reshape(1, -1))
    return out


# speedup vs baseline: 8.0442x; 1.1666x over previous
import functools

import jax
import jax.numpy as jnp
from jax import lax
from jax.experimental import pallas as pl
from jax.experimental.pallas import tpu as pltpu
from jax.experimental.pallas import tpu_sc as plsc

NC = 2
NS = 16
NW = NC * NS

CHUNK = 80


def _sc_scatter_add(n_nodes, n_edges, d):
    ept = n_edges // NW
    iters = ept // CHUNK
    assert iters >= 6
    rows_per_tile = -(-n_nodes // (8 * NS)) * 8
    n_pad = rows_per_tile * NS

    mesh = plsc.VectorSubcoreMesh(core_axis_name="c", subcore_axis_name="s")

    def body(dst_hbm, ea_hbm, out_hbm,
             idx0, idx1, idx2, row0, row1, row2, acc,
             g0, g1, g2, s0, s1, s2):
        c = lax.axis_index("c")
        s = lax.axis_index("s")
        wid = c * NS + s
        base = wid * ept
        idxs = (idx0, idx1, idx2)
        rows = (row0, row1, row2)
        gsem = (g0, g1, g2)
        ssem = (s0, s1, s2)

        def zrow(r, carry):
            z = jnp.zeros((16,), jnp.float32)
            for cc in range(d // 16):
                row0[r, pl.ds(cc * 16, 16)] = z
            return carry
        lax.fori_loop(0, CHUNK, zrow, 0)
        r0 = s * rows_per_tile
        for j in range(rows_per_tile // CHUNK):
            pltpu.sync_copy(row0, acc.at[pl.ds(r0 + j * CHUNK, CHUNK)])
        rem_rows = rows_per_tile % CHUNK
        if rem_rows:
            pltpu.sync_copy(row0.at[pl.ds(0, rem_rows)],
                            acc.at[pl.ds(r0 + rows_per_tile - rem_rows,
                                         rem_rows)])
        plsc.subcore_barrier()

        def issue_gather(i, b):
            off = base + i * CHUNK
            pltpu.async_copy(dst_hbm.at[pl.ds(off, CHUNK)], idxs[b], gsem[b])
            pltpu.async_copy(ea_hbm.at[pl.ds(off, CHUNK)], rows[b], gsem[b])

        def wait_gather(b):
            pltpu.make_async_copy(dst_hbm.at[pl.ds(0, CHUNK)], idxs[b],
                                  gsem[b]).wait()
            pltpu.make_async_copy(ea_hbm.at[pl.ds(0, CHUNK)], rows[b],
                                  gsem[b]).wait()

        def issue_scatter(b):
            pltpu.async_copy(rows[b], acc.at[idxs[b]], ssem[b], add=True)

        def wait_scatter(b):
            pltpu.make_async_copy(rows[b], acc.at[idxs[b]], ssem[b]).wait()

        issue_gather(0, 0)
        issue_gather(1, 1)
        wait_gather(0)
        issue_scatter(0)
        issue_gather(2, 2)
        wait_gather(1)
        issue_scatter(1)
        wait_scatter(0)
        issue_gather(3, 0)
        wait_gather(2)
        issue_scatter(2)
        wait_scatter(1)
        issue_gather(4, 1)

        G = iters // 3

        def group(g, carry):
            i0 = 3 * g
            for k in range(3):
                b = k
                nb = (k + 2) % 3
                wait_gather(b)
                issue_scatter(b)
                wait_scatter(nb)
                issue_gather(i0 + k + 2, nb)
            return carry
        lax.fori_loop(1, G, group, 0)

        for i in range(3 * G, iters):
            b = i % 3
            wait_gather(b)
            issue_scatter(b)
            wait_scatter((b + 2) % 3)
        wait_scatter((iters - 1) % 3)

        plsc.subcore_barrier()
        pltpu.sync_copy(acc.at[pl.ds(r0, rows_per_tile)],
                        out_hbm.at[c, pl.ds(r0, rows_per_tile)])

    scatter_kernel = functools.partial(
        pl.kernel,
        out_type=jax.ShapeDtypeStruct((NC, n_pad, d), jnp.float32),
        mesh=mesh,
        scratch_types=[
            pltpu.VMEM((CHUNK,), jnp.int32),
            pltpu.VMEM((CHUNK,), jnp.int32),
            pltpu.VMEM((CHUNK,), jnp.int32),
            pltpu.VMEM((CHUNK, d), jnp.float32),
            pltpu.VMEM((CHUNK, d), jnp.float32),
            pltpu.VMEM((CHUNK, d), jnp.float32),
            pltpu.VMEM_SHARED((n_pad, d), jnp.float32),
            pltpu.SemaphoreType.DMA,
            pltpu.SemaphoreType.DMA,
            pltpu.SemaphoreType.DMA,
            pltpu.SemaphoreType.DMA,
            pltpu.SemaphoreType.DMA,
            pltpu.SemaphoreType.DMA,
        ],
    )(body)
    return scatter_kernel, n_pad


def _mlp_body(x_ref, a2_ref, w1_ref, b1_ref,
              w2_ref, b2_ref, g_ref, bt_ref, o_ref):
    x = x_ref[...]
    a = a2_ref[0] + a2_ref[1]
    xa = jnp.concatenate([x, a], axis=-1)
    h = (jnp.dot(xa, w1_ref[...], preferred_element_type=jnp.float32,
                 precision=lax.Precision.DEFAULT)
         + b1_ref[...])
    h = h * jax.nn.sigmoid(h)
    y = (jnp.dot(h, w2_ref[...], preferred_element_type=jnp.float32,
                 precision=lax.Precision.DEFAULT)
         + b2_ref[...])
    mean = jnp.mean(y, axis=-1, keepdims=True)
    cen = y - mean
    var = jnp.mean(cen * cen, axis=-1, keepdims=True)
    o_ref[...] = cen * lax.rsqrt(var + 1e-5) * g_ref[...] + bt_ref[...] + x


def kernel(x, edge_index, edge_attr, W1, b1, W2, b2, ln_gamma, ln_beta):
    n_nodes, d = x.shape
    n_edges = edge_attr.shape[0]

    dst = jnp.asarray(edge_index[0], jnp.int32)

    sc_fn, n_pad = _sc_scatter_add(n_nodes, n_edges, d)
    agg2 = sc_fn(dst, edge_attr)

    blk = 1000
    grid = (n_nodes // blk,)
    row_spec = pl.BlockSpec((blk, d), lambda i: (i, 0))
    full = lambda shape: pl.BlockSpec(shape, lambda i: (0,) * len(shape))

    out = pl.pallas_call(
        _mlp_body,
        grid=grid,
        in_specs=[
            row_spec,
            pl.BlockSpec((2, blk, d), lambda i: (0, i, 0)),
            full((2 * d, d)), full((1, d)),
            full((d, d)), full((1, d)), full((1, d)), full((1, d)),
        ],
        out_specs=row_spec,
        out_shape=jax.ShapeDtypeStruct((n_nodes, d), jnp.float32),
    )(x, agg2, W1, b1.reshape(1, -1), W2, b2.reshape(1, -1),
      ln_gamma.reshape(1, -1), ln_beta.reshape(1, -1))
    return out


# speedup vs baseline: 8.1204x; 1.0095x over previous
import functools

import jax
import jax.numpy as jnp
from jax import lax
from jax.experimental import pallas as pl
from jax.experimental.pallas import tpu as pltpu
from jax.experimental.pallas import tpu_sc as plsc

NC = 2
NS = 16
NW = NC * NS

CHUNK = 80


def _sc_scatter_add(n_nodes, n_edges, d):
    ept = n_edges // NW
    iters = ept // CHUNK
    assert iters >= 6
    rows_per_tile = -(-n_nodes // (8 * NS)) * 8
    n_pad = rows_per_tile * NS

    mesh = plsc.VectorSubcoreMesh(core_axis_name="c", subcore_axis_name="s")

    def body(dst_hbm, ea_hbm, out_hbm,
             idx0, idx1, idx2, row0, row1, row2, acc,
             g0, g1, g2, s0, s1, s2):
        c = lax.axis_index("c")
        s = lax.axis_index("s")
        wid = c * NS + s
        base = wid * ept
        idxs = (idx0, idx1, idx2)
        rows = (row0, row1, row2)
        gsem = (g0, g1, g2)
        ssem = (s0, s1, s2)

        def issue_gather(i, b):
            off = base + i * CHUNK
            pltpu.async_copy(dst_hbm.at[pl.ds(off, CHUNK)], idxs[b], gsem[b])
            pltpu.async_copy(ea_hbm.at[pl.ds(off, CHUNK)], rows[b], gsem[b])

        def wait_gather(b):
            pltpu.make_async_copy(dst_hbm.at[pl.ds(0, CHUNK)], idxs[b],
                                  gsem[b]).wait()
            pltpu.make_async_copy(ea_hbm.at[pl.ds(0, CHUNK)], rows[b],
                                  gsem[b]).wait()

        def issue_scatter(b):
            pltpu.async_copy(rows[b], acc.at[idxs[b]], ssem[b], add=True)

        def wait_scatter(b):
            pltpu.make_async_copy(rows[b], acc.at[idxs[b]], ssem[b]).wait()

        issue_gather(0, 0)
        issue_gather(1, 1)

        def zrow(r, carry):
            z = jnp.zeros((16,), jnp.float32)
            for cc in range(d // 16):
                row2[r, pl.ds(cc * 16, 16)] = z
            return carry
        lax.fori_loop(0, CHUNK, zrow, 0)
        r0 = s * rows_per_tile
        for j in range(rows_per_tile // CHUNK):
            pltpu.sync_copy(row2, acc.at[pl.ds(r0 + j * CHUNK, CHUNK)])
        rem_rows = rows_per_tile % CHUNK
        if rem_rows:
            pltpu.sync_copy(row2.at[pl.ds(0, rem_rows)],
                            acc.at[pl.ds(r0 + rows_per_tile - rem_rows,
                                         rem_rows)])
        plsc.subcore_barrier()

        wait_gather(0)
        issue_scatter(0)
        issue_gather(2, 2)
        wait_gather(1)
        issue_scatter(1)
        wait_scatter(0)
        issue_gather(3, 0)
        wait_gather(2)
        issue_scatter(2)
        wait_scatter(1)
        issue_gather(4, 1)

        G = iters // 3

        def group(g, carry):
            i0 = 3 * g
            for k in range(3):
                b = k
                nb = (k + 2) % 3
                wait_gather(b)
                issue_scatter(b)
                wait_scatter(nb)
                issue_gather(i0 + k + 2, nb)
            return carry
        lax.fori_loop(1, G, group, 0)

        for i in range(3 * G, iters):
            b = i % 3
            wait_gather(b)
            issue_scatter(b)
            wait_scatter((b + 2) % 3)
        wait_scatter((iters - 1) % 3)

        plsc.subcore_barrier()
        pltpu.sync_copy(acc.at[pl.ds(r0, rows_per_tile)],
                        out_hbm.at[c, pl.ds(r0, rows_per_tile)])

    scatter_kernel = functools.partial(
        pl.kernel,
        out_type=jax.ShapeDtypeStruct((NC, n_pad, d), jnp.float32),
        mesh=mesh,
        scratch_types=[
            pltpu.VMEM((CHUNK,), jnp.int32),
            pltpu.VMEM((CHUNK,), jnp.int32),
            pltpu.VMEM((CHUNK,), jnp.int32),
            pltpu.VMEM((CHUNK, d), jnp.float32),
            pltpu.VMEM((CHUNK, d), jnp.float32),
            pltpu.VMEM((CHUNK, d), jnp.float32),
            pltpu.VMEM_SHARED((n_pad, d), jnp.float32),
            pltpu.SemaphoreType.DMA,
            pltpu.SemaphoreType.DMA,
            pltpu.SemaphoreType.DMA,
            pltpu.SemaphoreType.DMA,
            pltpu.SemaphoreType.DMA,
            pltpu.SemaphoreType.DMA,
        ],
    )(body)
    return scatter_kernel, n_pad


def _mlp_body(x_ref, a2_ref, w1_ref, b1_ref,
              w2_ref, b2_ref, g_ref, bt_ref, o_ref):
    x = x_ref[...]
    a = a2_ref[0] + a2_ref[1]
    xa = jnp.concatenate([x, a], axis=-1)
    h = (jnp.dot(xa, w1_ref[...], preferred_element_type=jnp.float32,
                 precision=lax.Precision.DEFAULT)
         + b1_ref[...])
    h = h * jax.nn.sigmoid(h)
    y = (jnp.dot(h, w2_ref[...], preferred_element_type=jnp.float32,
                 precision=lax.Precision.DEFAULT)
         + b2_ref[...])
    mean = jnp.mean(y, axis=-1, keepdims=True)
    cen = y - mean
    var = jnp.mean(cen * cen, axis=-1, keepdims=True)
    o_ref[...] = cen * lax.rsqrt(var + 1e-5) * g_ref[...] + bt_ref[...] + x


def kernel(x, edge_index, edge_attr, W1, b1, W2, b2, ln_gamma, ln_beta):
    n_nodes, d = x.shape
    n_edges = edge_attr.shape[0]

    dst = jnp.asarray(edge_index[0], jnp.int32)

    sc_fn, n_pad = _sc_scatter_add(n_nodes, n_edges, d)
    agg2 = sc_fn(dst, edge_attr)

    blk = 1000
    grid = (n_nodes // blk,)
    row_spec = pl.BlockSpec((blk, d), lambda i: (i, 0))
    full = lambda shape: pl.BlockSpec(shape, lambda i: (0,) * len(shape))

    out = pl.pallas_call(
        _mlp_body,
        grid=grid,
        in_specs=[
            row_spec,
            pl.BlockSpec((2, blk, d), lambda i: (0, i, 0)),
            full((2 * d, d)), full((1, d)),
            full((d, d)), full((1, d)), full((1, d)), full((1, d)),
        ],
        out_specs=row_spec,
        out_shape=jax.ShapeDtypeStruct((n_nodes, d), jnp.float32),
    )(x, agg2, W1, b1.reshape(1, -1), W2, b2.reshape(1, -1),
      ln_gamma.reshape(1, -1), ln_beta.reshape(1, -1))
    return out


# speedup vs baseline: 8.5686x; 1.0552x over previous
import functools

import jax
import jax.numpy as jnp
from jax import lax
from jax.experimental import pallas as pl
from jax.experimental.pallas import tpu as pltpu
from jax.experimental.pallas import tpu_sc as plsc

NC = 2
NS = 16
NW = NC * NS

CHUNK = 128


def _sc_scatter_add(n_nodes, n_edges, d):
    n_chunks = n_edges // CHUNK
    assert n_chunks * CHUNK == n_edges
    iters = n_chunks // NW
    extras = n_chunks % NW
    assert iters % 3 == 0 and iters >= 9
    rows_per_tile = -(-n_nodes // (8 * NS)) * 8
    n_pad = rows_per_tile * NS

    mesh = plsc.VectorSubcoreMesh(core_axis_name="c", subcore_axis_name="s")

    def body(ei_hbm, ea_hbm, out_hbm,
             idx0, idx1, idx2, row0, row1, row2, acc,
             g0, g1, g2, s0, s1, s2):
        c = lax.axis_index("c")
        s = lax.axis_index("s")
        wid = c * NS + s
        base = wid * iters
        idxs = (idx0, idx1, idx2)
        rows = (row0, row1, row2)
        gsem = (g0, g1, g2)
        ssem = (s0, s1, s2)

        def issue_gather_chunk(chunk, b):
            off = chunk * CHUNK
            pltpu.async_copy(ei_hbm.at[0, pl.ds(off, CHUNK)], idxs[b], gsem[b])
            pltpu.async_copy(ea_hbm.at[pl.ds(off, CHUNK)], rows[b], gsem[b])

        def issue_gather(i, b):
            issue_gather_chunk(base + i, b)

        def wait_gather(b):
            pltpu.make_async_copy(ei_hbm.at[0, pl.ds(0, CHUNK)], idxs[b],
                                  gsem[b]).wait()
            pltpu.make_async_copy(ea_hbm.at[pl.ds(0, CHUNK)], rows[b],
                                  gsem[b]).wait()

        def issue_scatter(b):
            pltpu.async_copy(rows[b], acc.at[idxs[b]], ssem[b], add=True)

        def wait_scatter(b):
            pltpu.make_async_copy(rows[b], acc.at[idxs[b]], ssem[b]).wait()

        issue_gather(0, 0)
        issue_gather(1, 1)

        def zrow(r, carry):
            z = jnp.zeros((16,), jnp.float32)
            for cc in range(d // 16):
                row2[r, pl.ds(cc * 16, 16)] = z
            return carry
        lax.fori_loop(0, CHUNK, zrow, 0)
        r0 = s * rows_per_tile
        for j in range(rows_per_tile // CHUNK):
            pltpu.sync_copy(row2, acc.at[pl.ds(r0 + j * CHUNK, CHUNK)])
        rem_rows = rows_per_tile % CHUNK
        if rem_rows:
            pltpu.sync_copy(row2.at[pl.ds(0, rem_rows)],
                            acc.at[pl.ds(r0 + rows_per_tile - rem_rows,
                                         rem_rows)])
        plsc.subcore_barrier()

        wait_gather(0)
        issue_scatter(0)
        issue_gather(2, 2)
        wait_gather(1)
        issue_scatter(1)
        wait_scatter(0)
        issue_gather(3, 0)
        wait_gather(2)
        issue_scatter(2)
        wait_scatter(1)
        issue_gather(4, 1)

        G = iters // 3

        def group(g, carry):
            i0 = 3 * g
            for k in range(3):
                b = k
                nb = (k + 2) % 3
                wait_gather(b)
                issue_scatter(b)
                wait_scatter(nb)
                issue_gather(i0 + k + 2, nb)
            return carry
        lax.fori_loop(1, G - 1, group, 0)

        wait_gather(0)
        issue_scatter(0)
        wait_scatter(2)
        issue_gather(iters - 1, 2)
        wait_gather(1)
        issue_scatter(1)
        wait_scatter(0)
        if extras:
            @pl.when(wid < extras)
            def _():
                issue_gather_chunk(NW * iters + wid, 0)
        wait_gather(2)
        issue_scatter(2)
        wait_scatter(1)
        if extras:
            @pl.when(wid < extras)
            def _():
                wait_gather(0)
                issue_scatter(0)
        wait_scatter(2)
        if extras:
            @pl.when(wid < extras)
            def _():
                wait_scatter(0)

        plsc.subcore_barrier()
        pltpu.sync_copy(acc.at[pl.ds(r0, rows_per_tile)],
                        out_hbm.at[c, pl.ds(r0, rows_per_tile)])

    scatter_kernel = functools.partial(
        pl.kernel,
        out_type=jax.ShapeDtypeStruct((NC, n_pad, d), jnp.float32),
        mesh=mesh,
        scratch_types=[
            pltpu.VMEM((CHUNK,), jnp.int32),
            pltpu.VMEM((CHUNK,), jnp.int32),
            pltpu.VMEM((CHUNK,), jnp.int32),
            pltpu.VMEM((CHUNK, d), jnp.float32),
            pltpu.VMEM((CHUNK, d), jnp.float32),
            pltpu.VMEM((CHUNK, d), jnp.float32),
            pltpu.VMEM_SHARED((n_pad, d), jnp.float32),
            pltpu.SemaphoreType.DMA,
            pltpu.SemaphoreType.DMA,
            pltpu.SemaphoreType.DMA,
            pltpu.SemaphoreType.DMA,
            pltpu.SemaphoreType.DMA,
            pltpu.SemaphoreType.DMA,
        ],
    )(body)
    return scatter_kernel, n_pad


def _mlp_body(x_ref, a2_ref, w1_ref, b1_ref,
              w2_ref, b2_ref, g_ref, bt_ref, o_ref):
    x = x_ref[...]
    a = a2_ref[0] + a2_ref[1]
    xa = jnp.concatenate([x, a], axis=-1)
    h = (jnp.dot(xa, w1_ref[...], preferred_element_type=jnp.float32,
                 precision=lax.Precision.DEFAULT)
         + b1_ref[...])
    h = h * jax.nn.sigmoid(h)
    y = (jnp.dot(h, w2_ref[...], preferred_element_type=jnp.float32,
                 precision=lax.Precision.DEFAULT)
         + b2_ref[...])
    mean = jnp.mean(y, axis=-1, keepdims=True)
    cen = y - mean
    var = jnp.mean(cen * cen, axis=-1, keepdims=True)
    o_ref[...] = cen * lax.rsqrt(var + 1e-5) * g_ref[...] + bt_ref[...] + x


def kernel(x, edge_index, edge_attr, W1, b1, W2, b2, ln_gamma, ln_beta):
    n_nodes, d = x.shape
    n_edges = edge_attr.shape[0]

    ei = jnp.asarray(edge_index, jnp.int32)

    sc_fn, n_pad = _sc_scatter_add(n_nodes, n_edges, d)
    agg2 = sc_fn(ei, edge_attr)

    blk = 2000
    grid = (n_nodes // blk,)
    row_spec = pl.BlockSpec((blk, d), lambda i: (i, 0))
    full = lambda shape: pl.BlockSpec(shape, lambda i: (0,) * len(shape))

    out = pl.pallas_call(
        _mlp_body,
        grid=grid,
        in_specs=[
            row_spec,
            pl.BlockSpec((2, blk, d), lambda i: (0, i, 0)),
            full((2 * d, d)), full((1, d)),
            full((d, d)), full((1, d)), full((1, d)), full((1, d)),
        ],
        out_specs=row_spec,
        out_shape=jax.ShapeDtypeStruct((n_nodes, d), jnp.float32),
    )(x, agg2, W1, b1.reshape(1, -1), W2, b2.reshape(1, -1),
      ln_gamma.reshape(1, -1), ln_beta.reshape(1, -1))
    return out
